# baseline (speedup 1.0000x reference)
import jax
import jax.numpy as jnp
from jax import lax
from jax.experimental import pallas as pl
from jax.experimental.pallas import tpu as pltpu
from jax.experimental.pallas import tpu_sc as plsc

N = 10000
E = 320000
D = 128
NC = 2
NS = 16
NW = NC * NS
EW = E // NW
C = 125
NCHUNK = EW // C
G = 16
NG = NCHUNK // G
NP = 10112
RPT = NP // NS
CW = 128

_mesh = plsc.VectorSubcoreMesh(core_axis_name="c", subcore_axis_name="s")


def _sc_agg_body(h, srcs, dsts, zrows, out_p, srcv, dstv, rows0, rows1, acc,
                 gsem, ssem):
  cid = lax.axis_index("c")
  sid = lax.axis_index("s")
  wid = cid * NS + sid
  pltpu.sync_copy(zrows.at[pl.ds(sid * RPT, RPT)], acc.at[pl.ds(sid * RPT, RPT)])
  plsc.subcore_barrier()
  rows = (rows0, rows1)

  def group(g, carry):
    pltpu.sync_copy(srcs.at[pl.ds(wid * NCHUNK + g * G, G)], srcv)
    pltpu.sync_copy(dsts.at[pl.ds(wid * NCHUNK + g * G, G)], dstv)
    scp = [None] * G
    gcp = pltpu.async_copy(h.at[srcv.at[0]], rows[0], gsem)
    for j in range(G):
      gcp.wait()
      scp[j] = pltpu.async_copy(rows[j % 2], acc.at[dstv.at[j]], ssem,
                                add=True)
      if j + 1 < G:
        if j >= 1:
          scp[j - 1].wait()
        gcp = pltpu.async_copy(h.at[srcv.at[j + 1]], rows[(j + 1) % 2], gsem)
    if G >= 2:
      scp[G - 2].wait()
    scp[G - 1].wait()
    return carry

  lax.fori_loop(0, NG, group, 0)
  plsc.subcore_barrier()
  pltpu.sync_copy(acc.at[pl.ds(sid * RPT, RPT)],
                  out_p.at[cid, pl.ds(sid * RPT, RPT)])


_sc_agg = pl.kernel(
    _sc_agg_body,
    out_type=jax.ShapeDtypeStruct((NC, NP, D), jnp.float32),
    mesh=_mesh,
    scratch_types=[
        pltpu.VMEM((G, C), jnp.int32),
        pltpu.VMEM((G, C), jnp.int32),
        pltpu.VMEM((C, D), jnp.float32),
        pltpu.VMEM((C, D), jnp.float32),
        pltpu.VMEM_SHARED((NP, D), jnp.float32),
        pltpu.SemaphoreType.DMA,
        pltpu.SemaphoreType.DMA,
    ],
)


def _sc_count_body(dsts, zcnt, ones, out_c, dstv, onesv, cacc, ssem):
  cid = lax.axis_index("c")
  sid = lax.axis_index("s")
  wid = cid * NS + sid
  pltpu.sync_copy(ones, onesv)
  pltpu.sync_copy(zcnt.at[pl.ds(sid * RPT, RPT)],
                  cacc.at[pl.ds(sid * RPT, RPT)])
  plsc.subcore_barrier()

  def group(g, carry):
    pltpu.sync_copy(dsts.at[pl.ds(wid * NCHUNK + g * G, G)], dstv)
    scp = [pltpu.async_copy(onesv, cacc.at[dstv.at[j]], ssem, add=True)
           for j in range(G)]
    for cp in scp:
      cp.wait()
    return carry

  lax.fori_loop(0, NG, group, 0)
  plsc.subcore_barrier()
  pltpu.sync_copy(cacc.at[pl.ds(sid * RPT, RPT)],
                  out_c.at[cid, pl.ds(sid * RPT, RPT)])


_sc_count = pl.kernel(
    _sc_count_body,
    out_type=jax.ShapeDtypeStruct((NC, NP, CW), jnp.float32),
    mesh=_mesh,
    scratch_types=[
        pltpu.VMEM((G, C), jnp.int32),
        pltpu.VMEM((C, CW), jnp.float32),
        pltpu.VMEM_SHARED((NP, CW), jnp.float32),
        pltpu.SemaphoreType.DMA,
    ],
)

_R = 400


def _make_tc_layer(act):
  def body(p0, p1, c0, c1, h, wl, wr, bias, out):
    cnt = c0[:, 0:1] + c1[:, 0:1]
    inv = 1.0 / jnp.maximum(cnt, 1.0)
    agg = (p0[...] + p1[...]) * inv
    y = (jnp.dot(agg, wl[...], preferred_element_type=jnp.float32)
         + jnp.dot(h[...], wr[...], preferred_element_type=jnp.float32)
         + bias[...])
    if act == "relu":
      out[...] = jnp.maximum(y, 0.0)
    else:
      out[...] = jnp.where(y > 0.0, y, jnp.exp(jnp.minimum(y, 0.0)) - 1.0)

  row_blk = pl.BlockSpec((_R, D), lambda i: (i, 0))
  cnt_blk = pl.BlockSpec((_R, CW), lambda i: (i, 0))
  full = pl.BlockSpec((D, D), lambda i: (0, 0))
  bias_blk = pl.BlockSpec((1, D), lambda i: (0, 0))
  return pl.pallas_call(
      body,
      grid=(N // _R,),
      in_specs=[row_blk, row_blk, cnt_blk, cnt_blk, row_blk, full, full,
                bias_blk],
      out_specs=row_blk,
      out_shape=jax.ShapeDtypeStruct((N, D), jnp.float32),
  )


_tc_relu = _make_tc_layer("relu")
_tc_elu = _make_tc_layer("elu")


def kernel(x, edge_index, edge_feats,
           Wl_0_0, bl_0_0, Wr_0_0, Wl_0_1, bl_0_1, Wr_0_1,
           Wl_1_0, bl_1_0, Wr_1_0, Wl_1_1, bl_1_1, Wr_1_1):
  src2 = edge_index[0].reshape(NW * NCHUNK, C)
  dst2 = edge_index[1].reshape(NW * NCHUNK, C)
  zrows = jnp.zeros((NP, D), jnp.float32)
  zcnt = jnp.zeros((NP, CW), jnp.float32)
  ones = jnp.ones((C, CW), jnp.float32)

  layers = [
      (Wl_0_0, bl_0_0, Wr_0_0, _tc_relu),
      (Wl_0_1, bl_0_1, Wr_0_1, _tc_elu),
      (Wl_1_0, bl_1_0, Wr_1_0, _tc_relu),
      (Wl_1_1, bl_1_1, Wr_1_1, _tc_elu),
  ]

  c = _sc_count(dst2, zcnt, ones)
  c0, c1 = c[0], c[1]
  h = x
  for wl, bias, wr, tc in layers:
    p = _sc_agg(h, src2, dst2, zrows)
    h = tc(p[0], p[1], c0, c1, h, wl, wr, bias.reshape(1, D))
  return h


# speedup vs baseline: 1.0149x; 1.0149x over previous
import jax
import jax.numpy as jnp
from jax import lax
from jax.experimental import pallas as pl
from jax.experimental.pallas import tpu as pltpu
from jax.experimental.pallas import tpu_sc as plsc

N = 10000
E = 320000
D = 128
NC = 2
NS = 16
NW = NC * NS
EW = E // NW
C = 125
NCHUNK = EW // C
G = 16
NG = NCHUNK // G
NP = 10112
RPT = NP // NS

_mesh = plsc.VectorSubcoreMesh(core_axis_name="c", subcore_axis_name="s")


def _zero_acc(zrows, acc, sid):
  pltpu.sync_copy(zrows.at[pl.ds(sid * RPT, RPT)], acc.at[pl.ds(sid * RPT, RPT)])


def _agg_chunks(h, srcs, dsts, acc, srcv, dstv, rows, gsem, ssem, wid):
  def group(g, carry):
    pltpu.sync_copy(srcs.at[pl.ds(wid * NCHUNK + g * G, G)], srcv)
    pltpu.sync_copy(dsts.at[pl.ds(wid * NCHUNK + g * G, G)], dstv)
    scp = [None] * G
    gcp = pltpu.async_copy(h.at[srcv.at[0]], rows[0], gsem)
    for j in range(G):
      gcp.wait()
      scp[j] = pltpu.async_copy(rows[j % 2], acc.at[dstv.at[j]], ssem,
                                add=True)
      if j + 1 < G:
        if j >= 1:
          scp[j - 1].wait()
        gcp = pltpu.async_copy(h.at[srcv.at[j + 1]], rows[(j + 1) % 2], gsem)
    scp[G - 2].wait()
    scp[G - 1].wait()
    return carry

  lax.fori_loop(0, NG, group, 0)


def _count_chunks(dsts, cacc, dstv, onesv, ssem, wid):
  def group(g, carry):
    pltpu.sync_copy(dsts.at[pl.ds(wid * NCHUNK + g * G, G)], dstv)
    scp = [pltpu.async_copy(onesv, cacc.at[dstv.at[j]], ssem, add=True)
           for j in range(G)]
    for cp in scp:
      cp.wait()
    return carry

  lax.fori_loop(0, NG, group, 0)


def _sc_agg0_body(h, srcs, dsts, zrows, ones, out_p, out_c, srcv, dstv,
                  rows0, rows1, acc, gsem, ssem):
  cid = lax.axis_index("c")
  sid = lax.axis_index("s")
  wid = cid * NS + sid
  _zero_acc(zrows, acc, sid)
  plsc.subcore_barrier()
  _agg_chunks(h, srcs, dsts, acc, srcv, dstv, (rows0, rows1), gsem, ssem, wid)
  plsc.subcore_barrier()
  pltpu.sync_copy(acc.at[pl.ds(sid * RPT, RPT)],
                  out_p.at[cid, pl.ds(sid * RPT, RPT)])
  _zero_acc(zrows, acc, sid)
  pltpu.sync_copy(ones, rows0)
  plsc.subcore_barrier()
  _count_chunks(dsts, acc, dstv, rows0, ssem, wid)
  plsc.subcore_barrier()
  pltpu.sync_copy(acc.at[pl.ds(sid * RPT, RPT)],
                  out_c.at[cid, pl.ds(sid * RPT, RPT)])


_sc_agg0 = pl.kernel(
    _sc_agg0_body,
    out_type=[jax.ShapeDtypeStruct((NC, NP, D), jnp.float32),
              jax.ShapeDtypeStruct((NC, NP, D), jnp.float32)],
    mesh=_mesh,
    scratch_types=[
        pltpu.VMEM((G, C), jnp.int32),
        pltpu.VMEM((G, C), jnp.int32),
        pltpu.VMEM((C, D), jnp.float32),
        pltpu.VMEM((C, D), jnp.float32),
        pltpu.VMEM_SHARED((NP, D), jnp.float32),
        pltpu.SemaphoreType.DMA,
        pltpu.SemaphoreType.DMA,
    ],
)


def _sc_agg_body(h, srcs, dsts, zrows, out_p, srcv, dstv, rows0, rows1, acc,
                 gsem, ssem):
  cid = lax.axis_index("c")
  sid = lax.axis_index("s")
  wid = cid * NS + sid
  _zero_acc(zrows, acc, sid)
  plsc.subcore_barrier()
  _agg_chunks(h, srcs, dsts, acc, srcv, dstv, (rows0, rows1), gsem, ssem, wid)
  plsc.subcore_barrier()
  pltpu.sync_copy(acc.at[pl.ds(sid * RPT, RPT)],
                  out_p.at[cid, pl.ds(sid * RPT, RPT)])


_sc_agg = pl.kernel(
    _sc_agg_body,
    out_type=jax.ShapeDtypeStruct((NC, NP, D), jnp.float32),
    mesh=_mesh,
    scratch_types=[
        pltpu.VMEM((G, C), jnp.int32),
        pltpu.VMEM((G, C), jnp.int32),
        pltpu.VMEM((C, D), jnp.float32),
        pltpu.VMEM((C, D), jnp.float32),
        pltpu.VMEM_SHARED((NP, D), jnp.float32),
        pltpu.SemaphoreType.DMA,
        pltpu.SemaphoreType.DMA,
    ],
)

_R = 400


def _act(act, y):
  if act == "relu":
    return jnp.maximum(y, 0.0)
  return jnp.where(y > 0.0, y, jnp.exp(jnp.minimum(y, 0.0)) - 1.0)


def _tc_layer0(p0, p1, c0, c1, h, wl, wr, bias):
  def body(p0r, p1r, c0r, c1r, hr, wlr, wrr, br, out, invout):
    cnt = c0r[:, 0:1] + c1r[:, 0:1]
    inv = 1.0 / jnp.maximum(cnt, 1.0)
    invb = jnp.broadcast_to(inv, (_R, D))
    agg = (p0r[...] + p1r[...]) * invb
    y = (jnp.dot(agg, wlr[...], preferred_element_type=jnp.float32)
         + jnp.dot(hr[...], wrr[...], preferred_element_type=jnp.float32)
         + br[...])
    out[...] = _act("relu", y)
    invout[...] = invb

  row_blk = pl.BlockSpec((_R, D), lambda i: (i, 0))
  full = pl.BlockSpec((D, D), lambda i: (0, 0))
  bias_blk = pl.BlockSpec((1, D), lambda i: (0, 0))
  return pl.pallas_call(
      body,
      grid=(N // _R,),
      in_specs=[row_blk, row_blk, row_blk, row_blk, row_blk, full, full,
                bias_blk],
      out_specs=[row_blk, row_blk],
      out_shape=[jax.ShapeDtypeStruct((N, D), jnp.float32),
                 jax.ShapeDtypeStruct((N, D), jnp.float32)],
  )(p0, p1, c0, c1, h, wl, wr, bias)


def _make_tc_layer(act):
  def body(p0r, p1r, invr, hr, wlr, wrr, br, out):
    agg = (p0r[...] + p1r[...]) * invr[...]
    y = (jnp.dot(agg, wlr[...], preferred_element_type=jnp.float32)
         + jnp.dot(hr[...], wrr[...], preferred_element_type=jnp.float32)
         + br[...])
    out[...] = _act(act, y)

  row_blk = pl.BlockSpec((_R, D), lambda i: (i, 0))
  full = pl.BlockSpec((D, D), lambda i: (0, 0))
  bias_blk = pl.BlockSpec((1, D), lambda i: (0, 0))
  return pl.pallas_call(
      body,
      grid=(N // _R,),
      in_specs=[row_blk, row_blk, row_blk, row_blk, full, full, bias_blk],
      out_specs=row_blk,
      out_shape=jax.ShapeDtypeStruct((N, D), jnp.float32),
  )


_tc_relu = _make_tc_layer("relu")
_tc_elu = _make_tc_layer("elu")


def kernel(x, edge_index, edge_feats,
           Wl_0_0, bl_0_0, Wr_0_0, Wl_0_1, bl_0_1, Wr_0_1,
           Wl_1_0, bl_1_0, Wr_1_0, Wl_1_1, bl_1_1, Wr_1_1):
  src2 = edge_index[0].reshape(NW * NCHUNK, C)
  dst2 = edge_index[1].reshape(NW * NCHUNK, C)
  zrows = jnp.zeros((NP, D), jnp.float32)
  ones = jnp.ones((C, D), jnp.float32)

  p, c = _sc_agg0(x, src2, dst2, zrows, ones)
  h, inv = _tc_layer0(p[0], p[1], c[0], c[1], x, Wl_0_0, Wr_0_0,
                      bl_0_0.reshape(1, D))

  for wl, bias, wr, tc in [
      (Wl_0_1, bl_0_1, Wr_0_1, _tc_elu),
      (Wl_1_0, bl_1_0, Wr_1_0, _tc_relu),
      (Wl_1_1, bl_1_1, Wr_1_1, _tc_elu),
  ]:
    p = _sc_agg(h, src2, dst2, zrows)
    h = tc(p[0], p[1], inv, h, wl, wr, bias.reshape(1, D))
  return h


# speedup vs baseline: 1.0165x; 1.0016x over previous
import jax
import jax.numpy as jnp
from jax import lax
from jax.experimental import pallas as pl
from jax.experimental.pallas import tpu as pltpu
from jax.experimental.pallas import tpu_sc as plsc

N = 10000
E = 320000
D = 128
NC = 2
NS = 16
NW = NC * NS
EW = E // NW
C = 125
NCHUNK = EW // C
G = 16
NG = NCHUNK // G
NP = 10112
RPT = NP // NS

_mesh = plsc.VectorSubcoreMesh(core_axis_name="c", subcore_axis_name="s")


def _zero_acc(zrows, acc, sid):
  pltpu.sync_copy(zrows.at[pl.ds(sid * RPT, RPT)], acc.at[pl.ds(sid * RPT, RPT)])


def _agg_chunks(h, srcs, dsts, acc, srcv, dstv, rows, gsem, ssem, wid):
  def group(g, carry):
    pltpu.sync_copy(srcs.at[pl.ds(wid * NCHUNK + g * G, G)], srcv)
    pltpu.sync_copy(dsts.at[pl.ds(wid * NCHUNK + g * G, G)], dstv)
    scp = [None] * G
    gcp = pltpu.async_copy(h.at[srcv.at[0]], rows[0], gsem)
    for j in range(G):
      gcp.wait()
      scp[j] = pltpu.async_copy(rows[j % 2], acc.at[dstv.at[j]], ssem,
                                add=True)
      if j + 1 < G:
        if j >= 1:
          scp[j - 1].wait()
        gcp = pltpu.async_copy(h.at[srcv.at[j + 1]], rows[(j + 1) % 2], gsem)
    scp[G - 2].wait()
    scp[G - 1].wait()
    return carry

  lax.fori_loop(0, NG, group, 0)


def _count_chunks(dsts, cacc, dstv, onesv, ssem, wid):
  def group(g, carry):
    pltpu.sync_copy(dsts.at[pl.ds(wid * NCHUNK + g * G, G)], dstv)
    scp = [pltpu.async_copy(onesv, cacc.at[dstv.at[j]], ssem, add=True)
           for j in range(G)]
    for cp in scp:
      cp.wait()
    return carry

  lax.fori_loop(0, NG, group, 0)


def _sc_agg0_body(h, srcs, dsts, zrows, ones, out_p, out_c, srcv, dstv,
                  rows0, rows1, acc, gsem, ssem):
  cid = lax.axis_index("c")
  sid = lax.axis_index("s")
  wid = cid * NS + sid
  _zero_acc(zrows, acc, sid)
  plsc.subcore_barrier()
  _agg_chunks(h, srcs, dsts, acc, srcv, dstv, (rows0, rows1), gsem, ssem, wid)
  plsc.subcore_barrier()
  pltpu.sync_copy(acc.at[pl.ds(sid * RPT, RPT)],
                  out_p.at[cid, pl.ds(sid * RPT, RPT)])
  _zero_acc(zrows, acc, sid)
  pltpu.sync_copy(ones, rows0)
  plsc.subcore_barrier()
  _count_chunks(dsts, acc, dstv, rows0, ssem, wid)
  plsc.subcore_barrier()
  pltpu.sync_copy(acc.at[pl.ds(sid * RPT, RPT)],
                  out_c.at[cid, pl.ds(sid * RPT, RPT)])


_sc_agg0 = pl.kernel(
    _sc_agg0_body,
    out_type=[jax.ShapeDtypeStruct((NC, NP, D), jnp.float32),
              jax.ShapeDtypeStruct((NC, NP, D), jnp.float32)],
    mesh=_mesh,
    scratch_types=[
        pltpu.VMEM((G, C), jnp.int32),
        pltpu.VMEM((G, C), jnp.int32),
        pltpu.VMEM((C, D), jnp.float32),
        pltpu.VMEM((C, D), jnp.float32),
        pltpu.VMEM_SHARED((NP, D), jnp.float32),
        pltpu.SemaphoreType.DMA,
        pltpu.SemaphoreType.DMA,
    ],
)


def _sc_agg_body(h, srcs, dsts, zrows, out_p, srcv, dstv, rows0, rows1, acc,
                 gsem, ssem):
  cid = lax.axis_index("c")
  sid = lax.axis_index("s")
  wid = cid * NS + sid
  _zero_acc(zrows, acc, sid)
  plsc.subcore_barrier()
  _agg_chunks(h, srcs, dsts, acc, srcv, dstv, (rows0, rows1), gsem, ssem, wid)
  plsc.subcore_barrier()
  pltpu.sync_copy(acc.at[pl.ds(sid * RPT, RPT)],
                  out_p.at[cid, pl.ds(sid * RPT, RPT)])


_sc_agg = pl.kernel(
    _sc_agg_body,
    out_type=jax.ShapeDtypeStruct((NC, NP, D), jnp.float32),
    mesh=_mesh,
    scratch_types=[
        pltpu.VMEM((G, C), jnp.int32),
        pltpu.VMEM((G, C), jnp.int32),
        pltpu.VMEM((C, D), jnp.float32),
        pltpu.VMEM((C, D), jnp.float32),
        pltpu.VMEM_SHARED((NP, D), jnp.float32),
        pltpu.SemaphoreType.DMA,
        pltpu.SemaphoreType.DMA,
    ],
)

_R = 400


def _act(act, y):
  if act == "relu":
    return jnp.maximum(y, 0.0)
  return jnp.where(y > 0.0, y, jnp.exp(jnp.minimum(y, 0.0)) - 1.0)


def _tc_pre(h, wr, bias):
  def body(hr, wrr, br, out):
    out[...] = jnp.dot(hr[...], wrr[...],
                       preferred_element_type=jnp.float32) + br[...]

  row_blk = pl.BlockSpec((_R, D), lambda i: (i, 0))
  full = pl.BlockSpec((D, D), lambda i: (0, 0))
  bias_blk = pl.BlockSpec((1, D), lambda i: (0, 0))
  return pl.pallas_call(
      body,
      grid=(N // _R,),
      in_specs=[row_blk, full, bias_blk],
      out_specs=row_blk,
      out_shape=jax.ShapeDtypeStruct((N, D), jnp.float32),
  )(h, wr, bias)


def _tc_layer0(p0, p1, c0, c1, r, wl):
  def body(p0r, p1r, c0r, c1r, rr, wlr, out, invout):
    cnt = c0r[:, 0:1] + c1r[:, 0:1]
    inv = 1.0 / jnp.maximum(cnt, 1.0)
    invb = jnp.broadcast_to(inv, (_R, D))
    agg = (p0r[...] + p1r[...]) * invb
    y = jnp.dot(agg, wlr[...], preferred_element_type=jnp.float32) + rr[...]
    out[...] = _act("relu", y)
    invout[...] = invb

  row_blk = pl.BlockSpec((_R, D), lambda i: (i, 0))
  full = pl.BlockSpec((D, D), lambda i: (0, 0))
  return pl.pallas_call(
      body,
      grid=(N // _R,),
      in_specs=[row_blk, row_blk, row_blk, row_blk, row_blk, full],
      out_specs=[row_blk, row_blk],
      out_shape=[jax.ShapeDtypeStruct((N, D), jnp.float32),
                 jax.ShapeDtypeStruct((N, D), jnp.float32)],
  )(p0, p1, c0, c1, r, wl)


def _make_tc_layer(act):
  def body(p0r, p1r, invr, rr, wlr, out):
    agg = (p0r[...] + p1r[...]) * invr[...]
    y = jnp.dot(agg, wlr[...], preferred_element_type=jnp.float32) + rr[...]
    out[...] = _act(act, y)

  row_blk = pl.BlockSpec((_R, D), lambda i: (i, 0))
  full = pl.BlockSpec((D, D), lambda i: (0, 0))
  return pl.pallas_call(
      body,
      grid=(N // _R,),
      in_specs=[row_blk, row_blk, row_blk, row_blk, full],
      out_specs=row_blk,
      out_shape=jax.ShapeDtypeStruct((N, D), jnp.float32),
  )


_tc_relu = _make_tc_layer("relu")
_tc_elu = _make_tc_layer("elu")


def kernel(x, edge_index, edge_feats,
           Wl_0_0, bl_0_0, Wr_0_0, Wl_0_1, bl_0_1, Wr_0_1,
           Wl_1_0, bl_1_0, Wr_1_0, Wl_1_1, bl_1_1, Wr_1_1):
  src2 = edge_index[0].reshape(NW * NCHUNK, C)
  dst2 = edge_index[1].reshape(NW * NCHUNK, C)
  zrows = jnp.zeros((NP, D), jnp.float32)
  ones = jnp.ones((C, D), jnp.float32)

  p, c = _sc_agg0(x, src2, dst2, zrows, ones)
  r = _tc_pre(x, Wr_0_0, bl_0_0.reshape(1, D))
  h, inv = _tc_layer0(p[0], p[1], c[0], c[1], r, Wl_0_0)

  for wl, bias, wr, tc in [
      (Wl_0_1, bl_0_1, Wr_0_1, _tc_elu),
      (Wl_1_0, bl_1_0, Wr_1_0, _tc_relu),
      (Wl_1_1, bl_1_1, Wr_1_1, _tc_elu),
  ]:
    p = _sc_agg(h, src2, dst2, zrows)
    r = _tc_pre(h, wr, bias.reshape(1, D))
    h = tc(p[0], p[1], inv, r, wl)
  return h


# speedup vs baseline: 1.1049x; 1.0870x over previous
import jax
import jax.numpy as jnp
from jax import lax
from jax.experimental import pallas as pl
from jax.experimental.pallas import tpu as pltpu
from jax.experimental.pallas import tpu_sc as plsc

N = 10000
E = 320000
D = 128
NC = 2
NS = 16
NW = NC * NS
EW = E // NW
C = 125
NCHUNK = EW // C
G = 40
NG = NCHUNK // G
NP = 10112
RPT = NP // NS

_mesh = plsc.VectorSubcoreMesh(core_axis_name="c", subcore_axis_name="s")


def _zero_acc(zrows, acc, sid):
  pltpu.sync_copy(zrows.at[pl.ds(sid * RPT, RPT)], acc.at[pl.ds(sid * RPT, RPT)])


def _agg_chunks(h, srcs, dsts, acc, srcv, dstv, rows, gsem, ssem, wid):
  def group(g, carry):
    pltpu.sync_copy(srcs.at[pl.ds(wid * NCHUNK + g * G, G)], srcv)
    pltpu.sync_copy(dsts.at[pl.ds(wid * NCHUNK + g * G, G)], dstv)
    scp = [None] * G
    gcp = pltpu.async_copy(h.at[srcv.at[0]], rows[0], gsem)
    for j in range(G):
      gcp.wait()
      scp[j] = pltpu.async_copy(rows[j % 2], acc.at[dstv.at[j]], ssem,
                                add=True)
      if j + 1 < G:
        if j >= 1:
          scp[j - 1].wait()
        gcp = pltpu.async_copy(h.at[srcv.at[j + 1]], rows[(j + 1) % 2], gsem)
    scp[G - 2].wait()
    scp[G - 1].wait()
    return carry

  lax.fori_loop(0, NG, group, 0)


def _count_chunks(dsts, cacc, dstv, onesv, ssem, wid):
  def group(g, carry):
    pltpu.sync_copy(dsts.at[pl.ds(wid * NCHUNK + g * G, G)], dstv)
    scp = [pltpu.async_copy(onesv, cacc.at[dstv.at[j]], ssem, add=True)
           for j in range(G)]
    for cp in scp:
      cp.wait()
    return carry

  lax.fori_loop(0, NG, group, 0)


def _sc_agg0_body(h, srcs, dsts, zrows, ones, out_p, out_c, srcv, dstv,
                  rows0, rows1, acc, gsem, ssem):
  cid = lax.axis_index("c")
  sid = lax.axis_index("s")
  wid = cid * NS + sid
  _zero_acc(zrows, acc, sid)
  plsc.subcore_barrier()
  _agg_chunks(h, srcs, dsts, acc, srcv, dstv, (rows0, rows1), gsem, ssem, wid)
  plsc.subcore_barrier()
  pltpu.sync_copy(acc.at[pl.ds(sid * RPT, RPT)],
                  out_p.at[cid, pl.ds(sid * RPT, RPT)])
  _zero_acc(zrows, acc, sid)
  pltpu.sync_copy(ones, rows0)
  plsc.subcore_barrier()
  _count_chunks(dsts, acc, dstv, rows0, ssem, wid)
  plsc.subcore_barrier()
  pltpu.sync_copy(acc.at[pl.ds(sid * RPT, RPT)],
                  out_c.at[cid, pl.ds(sid * RPT, RPT)])


_sc_agg0 = pl.kernel(
    _sc_agg0_body,
    out_type=[jax.ShapeDtypeStruct((NC, NP, D), jnp.float32),
              jax.ShapeDtypeStruct((NC, NP, D), jnp.float32)],
    mesh=_mesh,
    scratch_types=[
        pltpu.VMEM((G, C), jnp.int32),
        pltpu.VMEM((G, C), jnp.int32),
        pltpu.VMEM((C, D), jnp.float32),
        pltpu.VMEM((C, D), jnp.float32),
        pltpu.VMEM_SHARED((NP, D), jnp.float32),
        pltpu.SemaphoreType.DMA,
        pltpu.SemaphoreType.DMA,
    ],
)


def _sc_agg_body(h, srcs, dsts, zrows, out_p, srcv, dstv, rows0, rows1, acc,
                 gsem, ssem):
  cid = lax.axis_index("c")
  sid = lax.axis_index("s")
  wid = cid * NS + sid
  _zero_acc(zrows, acc, sid)
  plsc.subcore_barrier()
  _agg_chunks(h, srcs, dsts, acc, srcv, dstv, (rows0, rows1), gsem, ssem, wid)
  plsc.subcore_barrier()
  pltpu.sync_copy(acc.at[pl.ds(sid * RPT, RPT)],
                  out_p.at[cid, pl.ds(sid * RPT, RPT)])


_sc_agg = pl.kernel(
    _sc_agg_body,
    out_type=jax.ShapeDtypeStruct((NC, NP, D), jnp.float32),
    mesh=_mesh,
    scratch_types=[
        pltpu.VMEM((G, C), jnp.int32),
        pltpu.VMEM((G, C), jnp.int32),
        pltpu.VMEM((C, D), jnp.float32),
        pltpu.VMEM((C, D), jnp.float32),
        pltpu.VMEM_SHARED((NP, D), jnp.float32),
        pltpu.SemaphoreType.DMA,
        pltpu.SemaphoreType.DMA,
    ],
)

_R = 2000


def _act(act, y):
  if act == "relu":
    return jnp.maximum(y, 0.0)
  return jnp.where(y > 0.0, y, jnp.exp(jnp.minimum(y, 0.0)) - 1.0)


def _tc_pre(h, wr, bias):
  def body(hr, wrr, br, out):
    out[...] = jnp.dot(hr[...], wrr[...],
                       preferred_element_type=jnp.float32) + br[...]

  row_blk = pl.BlockSpec((_R, D), lambda i: (i, 0))
  full = pl.BlockSpec((D, D), lambda i: (0, 0))
  bias_blk = pl.BlockSpec((1, D), lambda i: (0, 0))
  return pl.pallas_call(
      body,
      grid=(N // _R,),
      in_specs=[row_blk, full, bias_blk],
      out_specs=row_blk,
      out_shape=jax.ShapeDtypeStruct((N, D), jnp.float32),
  )(h, wr, bias)


def _tc_layer0(p0, p1, c0, c1, r, wl):
  def body(p0r, p1r, c0r, c1r, rr, wlr, out, invout):
    cnt = c0r[:, 0:1] + c1r[:, 0:1]
    inv = 1.0 / jnp.maximum(cnt, 1.0)
    invb = jnp.broadcast_to(inv, (_R, D))
    agg = (p0r[...] + p1r[...]) * invb
    y = jnp.dot(agg, wlr[...], preferred_element_type=jnp.float32) + rr[...]
    out[...] = _act("relu", y)
    invout[...] = invb

  row_blk = pl.BlockSpec((_R, D), lambda i: (i, 0))
  full = pl.BlockSpec((D, D), lambda i: (0, 0))
  return pl.pallas_call(
      body,
      grid=(N // _R,),
      in_specs=[row_blk, row_blk, row_blk, row_blk, row_blk, full],
      out_specs=[row_blk, row_blk],
      out_shape=[jax.ShapeDtypeStruct((N, D), jnp.float32),
                 jax.ShapeDtypeStruct((N, D), jnp.float32)],
  )(p0, p1, c0, c1, r, wl)


def _make_tc_layer(act):
  def body(p0r, p1r, invr, rr, wlr, out):
    agg = (p0r[...] + p1r[...]) * invr[...]
    y = jnp.dot(agg, wlr[...], preferred_element_type=jnp.float32) + rr[...]
    out[...] = _act(act, y)

  row_blk = pl.BlockSpec((_R, D), lambda i: (i, 0))
  full = pl.BlockSpec((D, D), lambda i: (0, 0))
  return pl.pallas_call(
      body,
      grid=(N // _R,),
      in_specs=[row_blk, row_blk, row_blk, row_blk, full],
      out_specs=row_blk,
      out_shape=jax.ShapeDtypeStruct((N, D), jnp.float32),
  )


_tc_relu = _make_tc_layer("relu")
_tc_elu = _make_tc_layer("elu")


def kernel(x, edge_index, edge_feats,
           Wl_0_0, bl_0_0, Wr_0_0, Wl_0_1, bl_0_1, Wr_0_1,
           Wl_1_0, bl_1_0, Wr_1_0, Wl_1_1, bl_1_1, Wr_1_1):
  src2 = edge_index[0].reshape(NW * NCHUNK, C)
  dst2 = edge_index[1].reshape(NW * NCHUNK, C)
  zrows = jnp.zeros((NP, D), jnp.float32)
  ones = jnp.ones((C, D), jnp.float32)

  p, c = _sc_agg0(x, src2, dst2, zrows, ones)
  r = _tc_pre(x, Wr_0_0, bl_0_0.reshape(1, D))
  h, inv = _tc_layer0(p[0], p[1], c[0], c[1], r, Wl_0_0)

  for wl, bias, wr, tc in [
      (Wl_0_1, bl_0_1, Wr_0_1, _tc_elu),
      (Wl_1_0, bl_1_0, Wr_1_0, _tc_relu),
      (Wl_1_1, bl_1_1, Wr_1_1, _tc_elu),
  ]:
    p = _sc_agg(h, src2, dst2, zrows)
    r = _tc_pre(h, wr, bias.reshape(1, D))
    h = tc(p[0], p[1], inv, r, wl)
  return h


# speedup vs baseline: 1.1103x; 1.0049x over previous
import jax
import jax.numpy as jnp
from jax import lax
from jax.experimental import pallas as pl
from jax.experimental.pallas import tpu as pltpu
from jax.experimental.pallas import tpu_sc as plsc

N = 10000
E = 320000
D = 128
NC = 2
NS = 16
NW = NC * NS
EW = E // NW
C = 125
NCHUNK = EW // C
G = 40
NG = NCHUNK // G
NP = 10112
RPT = NP // NS

_mesh = plsc.VectorSubcoreMesh(core_axis_name="c", subcore_axis_name="s")


def _zero_acc(zrows, acc, sid):
  pltpu.sync_copy(zrows.at[pl.ds(sid * RPT, RPT)], acc.at[pl.ds(sid * RPT, RPT)])


def _agg_chunks(h, srcs, dsts, acc, srcv, dstv, rows, gsem, ssem, wid):
  pltpu.sync_copy(srcs.at[pl.ds(wid * NCHUNK, NCHUNK)], srcv)

  def group(g, carry):
    pltpu.sync_copy(dsts.at[pl.ds(wid * NCHUNK + g * G, G)], dstv)
    scp = [None] * G
    gcp = pltpu.async_copy(h.at[srcv.at[jnp.int32(0) + g * G]], rows[0], gsem)
    for j in range(G):
      gcp.wait()
      scp[j] = pltpu.async_copy(rows[j % 2], acc.at[dstv.at[j]], ssem,
                                add=True)
      if j + 1 < G:
        if j >= 1:
          scp[j - 1].wait()
        gcp = pltpu.async_copy(h.at[srcv.at[g * G + (j + 1)]],
                               rows[(j + 1) % 2], gsem)
    scp[G - 2].wait()
    scp[G - 1].wait()
    return carry

  lax.fori_loop(0, NG, group, 0)


def _count_chunks(dsts, cacc, dstv, onesv, ssem, wid):
  def group(g, carry):
    pltpu.sync_copy(dsts.at[pl.ds(wid * NCHUNK + g * G, G)], dstv)
    scp = [pltpu.async_copy(onesv, cacc.at[dstv.at[j]], ssem, add=True)
           for j in range(G)]
    for cp in scp:
      cp.wait()
    return carry

  lax.fori_loop(0, NG, group, 0)


def _sc_agg0_body(h, srcs, dsts, zrows, ones, out_p, out_c, srcv, dstv,
                  rows0, rows1, acc, gsem, ssem):
  cid = lax.axis_index("c")
  sid = lax.axis_index("s")
  wid = cid * NS + sid
  _zero_acc(zrows, acc, sid)
  plsc.subcore_barrier()
  _agg_chunks(h, srcs, dsts, acc, srcv, dstv, (rows0, rows1), gsem, ssem, wid)
  plsc.subcore_barrier()
  pltpu.sync_copy(acc.at[pl.ds(sid * RPT, RPT)],
                  out_p.at[cid, pl.ds(sid * RPT, RPT)])
  _zero_acc(zrows, acc, sid)
  pltpu.sync_copy(ones, rows0)
  plsc.subcore_barrier()
  _count_chunks(dsts, acc, dstv, rows0, ssem, wid)
  plsc.subcore_barrier()
  pltpu.sync_copy(acc.at[pl.ds(sid * RPT, RPT)],
                  out_c.at[cid, pl.ds(sid * RPT, RPT)])


_sc_agg0 = pl.kernel(
    _sc_agg0_body,
    out_type=[jax.ShapeDtypeStruct((NC, NP, D), jnp.float32),
              jax.ShapeDtypeStruct((NC, NP, D), jnp.float32)],
    mesh=_mesh,
    scratch_types=[
        pltpu.VMEM((NCHUNK, C), jnp.int32),
        pltpu.VMEM((G, C), jnp.int32),
        pltpu.VMEM((C, D), jnp.float32),
        pltpu.VMEM((C, D), jnp.float32),
        pltpu.VMEM_SHARED((NP, D), jnp.float32),
        pltpu.SemaphoreType.DMA,
        pltpu.SemaphoreType.DMA,
    ],
)


def _sc_agg_body(h, srcs, dsts, zrows, out_p, srcv, dstv, rows0, rows1, acc,
                 gsem, ssem):
  cid = lax.axis_index("c")
  sid = lax.axis_index("s")
  wid = cid * NS + sid
  _zero_acc(zrows, acc, sid)
  plsc.subcore_barrier()
  _agg_chunks(h, srcs, dsts, acc, srcv, dstv, (rows0, rows1), gsem, ssem, wid)
  plsc.subcore_barrier()
  pltpu.sync_copy(acc.at[pl.ds(sid * RPT, RPT)],
                  out_p.at[cid, pl.ds(sid * RPT, RPT)])


_sc_agg = pl.kernel(
    _sc_agg_body,
    out_type=jax.ShapeDtypeStruct((NC, NP, D), jnp.float32),
    mesh=_mesh,
    scratch_types=[
        pltpu.VMEM((NCHUNK, C), jnp.int32),
        pltpu.VMEM((G, C), jnp.int32),
        pltpu.VMEM((C, D), jnp.float32),
        pltpu.VMEM((C, D), jnp.float32),
        pltpu.VMEM_SHARED((NP, D), jnp.float32),
        pltpu.SemaphoreType.DMA,
        pltpu.SemaphoreType.DMA,
    ],
)

_R = 2000


def _act(act, y):
  if act == "relu":
    return jnp.maximum(y, 0.0)
  return jnp.where(y > 0.0, y, jnp.exp(jnp.minimum(y, 0.0)) - 1.0)


def _tc_pre(h, wr, bias):
  def body(hr, wrr, br, out):
    out[...] = jnp.dot(hr[...], wrr[...],
                       preferred_element_type=jnp.float32) + br[...]

  row_blk = pl.BlockSpec((_R, D), lambda i: (i, 0))
  full = pl.BlockSpec((D, D), lambda i: (0, 0))
  bias_blk = pl.BlockSpec((1, D), lambda i: (0, 0))
  return pl.pallas_call(
      body,
      grid=(N // _R,),
      in_specs=[row_blk, full, bias_blk],
      out_specs=row_blk,
      out_shape=jax.ShapeDtypeStruct((N, D), jnp.float32),
  )(h, wr, bias)


def _tc_layer0(p0, p1, c0, c1, r, wl):
  def body(p0r, p1r, c0r, c1r, rr, wlr, out, invout):
    cnt = c0r[:, 0:1] + c1r[:, 0:1]
    inv = 1.0 / jnp.maximum(cnt, 1.0)
    invb = jnp.broadcast_to(inv, (_R, D))
    agg = (p0r[...] + p1r[...]) * invb
    y = jnp.dot(agg, wlr[...], preferred_element_type=jnp.float32) + rr[...]
    out[...] = _act("relu", y)
    invout[...] = invb

  row_blk = pl.BlockSpec((_R, D), lambda i: (i, 0))
  full = pl.BlockSpec((D, D), lambda i: (0, 0))
  return pl.pallas_call(
      body,
      grid=(N // _R,),
      in_specs=[row_blk, row_blk, row_blk, row_blk, row_blk, full],
      out_specs=[row_blk, row_blk],
      out_shape=[jax.ShapeDtypeStruct((N, D), jnp.float32),
                 jax.ShapeDtypeStruct((N, D), jnp.float32)],
  )(p0, p1, c0, c1, r, wl)


def _make_tc_layer(act):
  def body(p0r, p1r, invr, rr, wlr, out):
    agg = (p0r[...] + p1r[...]) * invr[...]
    y = jnp.dot(agg, wlr[...], preferred_element_type=jnp.float32) + rr[...]
    out[...] = _act(act, y)

  row_blk = pl.BlockSpec((_R, D), lambda i: (i, 0))
  full = pl.BlockSpec((D, D), lambda i: (0, 0))
  return pl.pallas_call(
      body,
      grid=(N // _R,),
      in_specs=[row_blk, row_blk, row_blk, row_blk, full],
      out_specs=row_blk,
      out_shape=jax.ShapeDtypeStruct((N, D), jnp.float32),
  )


_tc_relu = _make_tc_layer("relu")
_tc_elu = _make_tc_layer("elu")


def kernel(x, edge_index, edge_feats,
           Wl_0_0, bl_0_0, Wr_0_0, Wl_0_1, bl_0_1, Wr_0_1,
           Wl_1_0, bl_1_0, Wr_1_0, Wl_1_1, bl_1_1, Wr_1_1):
  src2 = edge_index[0].reshape(NW * NCHUNK, C)
  dst2 = edge_index[1].reshape(NW * NCHUNK, C)
  zrows = jnp.zeros((NP, D), jnp.float32)
  ones = jnp.ones((C, D), jnp.float32)

  p, c = _sc_agg0(x, src2, dst2, zrows, ones)
  r = _tc_pre(x, Wr_0_0, bl_0_0.reshape(1, D))
  h, inv = _tc_layer0(p[0], p[1], c[0], c[1], r, Wl_0_0)

  for wl, bias, wr, tc in [
      (Wl_0_1, bl_0_1, Wr_0_1, _tc_elu),
      (Wl_1_0, bl_1_0, Wr_1_0, _tc_relu),
      (Wl_1_1, bl_1_1, Wr_1_1, _tc_elu),
  ]:
    p = _sc_agg(h, src2, dst2, zrows)
    r = _tc_pre(h, wr, bias.reshape(1, D))
    h = tc(p[0], p[1], inv, r, wl)
  return h
